# baseline (speedup 1.0000x reference)
import jax
import jax.numpy as jnp
from jax.experimental import pallas as pl
from jax.experimental.pallas import tpu as pltpu

_C = 1024
_NB = 8


def _make_body(T):
    def body(x_hbm, w_ref, b_ref, o_hbm, in_buf, out_buf, in_sem, out_sem):
        i = pl.program_id(0)

        def in_copy(chunk, slot):
            return pltpu.make_async_copy(
                x_hbm.at[pl.ds(chunk * _C, _C), :], in_buf.at[slot],
                in_sem.at[slot])

        def out_copy(chunk, slot):
            return pltpu.make_async_copy(
                out_buf.at[slot], o_hbm.at[pl.ds(chunk * _C, _C), :],
                out_sem.at[slot])

        @pl.when(i == 0)
        def _prologue():
            for j in range(_NB - 1):
                in_copy(j, j).start()

        slot = jax.lax.rem(i, _NB)

        nxt = i + _NB - 1

        @pl.when(nxt < T)
        def _prefetch():
            in_copy(nxt, jax.lax.rem(nxt, _NB)).start()

        @pl.when(i >= _NB)
        def _reclaim():
            out_copy(i - _NB, slot).wait()

        in_copy(i, slot).wait()
        acc = jnp.dot(in_buf[slot], w_ref[...],
                      preferred_element_type=jnp.float32)
        out_buf[slot] = jnp.tanh(acc + b_ref[...])
        out_copy(i, slot).start()

        @pl.when(i == T - 1)
        def _epilogue():
            for c in range(max(0, T - _NB), T):
                out_copy(c, c % _NB).wait()

    return body


def kernel(x, W, b):
    Bsz, N, K = x.shape
    M = Bsz * N
    T = M // _C
    x2 = x.reshape(M, K)
    b2 = b.reshape(1, K)
    out = pl.pallas_call(
        _make_body(T),
        grid=(T,),
        in_specs=[
            pl.BlockSpec(memory_space=pltpu.HBM),
            pl.BlockSpec((K, K), lambda i: (0, 0)),
            pl.BlockSpec((1, K), lambda i: (0, 0)),
        ],
        out_specs=pl.BlockSpec(memory_space=pltpu.HBM),
        out_shape=jax.ShapeDtypeStruct((M, K), jnp.float32),
        scratch_shapes=[
            pltpu.VMEM((_NB, _C, K), jnp.float32),
            pltpu.VMEM((_NB, _C, K), jnp.float32),
            pltpu.SemaphoreType.DMA((_NB,)),
            pltpu.SemaphoreType.DMA((_NB,)),
        ],
        compiler_params=pltpu.CompilerParams(
            dimension_semantics=("arbitrary",),
        ),
    )(x2, W, b2)
    return out.reshape(Bsz, N, K)


# speedup vs baseline: 1.0793x; 1.0793x over previous
import jax
import jax.numpy as jnp
from jax.experimental import pallas as pl
from jax.experimental.pallas import tpu as pltpu

_C = 2048
_NB = 4


def _make_body(T):
    def body(x_hbm, w_ref, b_ref, o_hbm, in_buf, out_buf, in_sem, out_sem):
        i = pl.program_id(0)

        def in_copy(chunk, slot):
            return pltpu.make_async_copy(
                x_hbm.at[pl.ds(chunk * _C, _C), :], in_buf.at[slot],
                in_sem.at[slot])

        def out_copy(chunk, slot):
            return pltpu.make_async_copy(
                out_buf.at[slot], o_hbm.at[pl.ds(chunk * _C, _C), :],
                out_sem.at[slot])

        @pl.when(i == 0)
        def _prologue():
            for j in range(_NB - 1):
                in_copy(j, j).start()

        slot = jax.lax.rem(i, _NB)

        nxt = i + _NB - 1

        @pl.when(nxt < T)
        def _prefetch():
            in_copy(nxt, jax.lax.rem(nxt, _NB)).start()

        @pl.when(i >= _NB)
        def _reclaim():
            out_copy(i - _NB, slot).wait()

        in_copy(i, slot).wait()
        acc = jnp.dot(in_buf[slot], w_ref[...],
                      preferred_element_type=jnp.float32)
        out_buf[slot] = jnp.tanh(acc + b_ref[...])
        out_copy(i, slot).start()

        @pl.when(i == T - 1)
        def _epilogue():
            for c in range(max(0, T - _NB), T):
                out_copy(c, c % _NB).wait()

    return body


def kernel(x, W, b):
    Bsz, N, K = x.shape
    M = Bsz * N
    T = M // _C
    x2 = x.reshape(M, K)
    b2 = b.reshape(1, K)
    out = pl.pallas_call(
        _make_body(T),
        grid=(T,),
        in_specs=[
            pl.BlockSpec(memory_space=pltpu.HBM),
            pl.BlockSpec((K, K), lambda i: (0, 0)),
            pl.BlockSpec((1, K), lambda i: (0, 0)),
        ],
        out_specs=pl.BlockSpec(memory_space=pltpu.HBM),
        out_shape=jax.ShapeDtypeStruct((M, K), jnp.float32),
        scratch_shapes=[
            pltpu.VMEM((_NB, _C, K), jnp.float32),
            pltpu.VMEM((_NB, _C, K), jnp.float32),
            pltpu.SemaphoreType.DMA((_NB,)),
            pltpu.SemaphoreType.DMA((_NB,)),
        ],
        compiler_params=pltpu.CompilerParams(
            dimension_semantics=("arbitrary",),
        ),
    )(x2, W, b2)
    return out.reshape(Bsz, N, K)
